# baseline (speedup 1.0000x reference)
import functools

import jax
import jax.numpy as jnp
from jax import lax
from jax.experimental import pallas as pl
from jax.experimental.pallas import tpu as pltpu
from jax.experimental.pallas import tpu_sc as plsc

_BATCH = 16384
_DIM = 16
_LANES = 16
_NUM_CORES = 2
_NUM_SUBCORES = 16
_NW = _NUM_CORES * _NUM_SUBCORES
_BPW = _BATCH // _NW
_GATHER = _BPW * _DIM

_STUDENT_ROWS = 1000000
_SUBJECT_ROWS = 100000

_mesh = plsc.VectorSubcoreMesh(core_axis_name="c", subcore_axis_name="s")


@functools.partial(
    pl.kernel,
    out_type=jax.ShapeDtypeStruct((_BATCH,), jnp.float32),
    mesh=_mesh,
    scratch_types=[
        pltpu.VMEM((_BPW,), jnp.int32),
        pltpu.VMEM((_BPW,), jnp.int32),
        pltpu.VMEM((_GATHER,), jnp.int32),
        pltpu.VMEM((_GATHER,), jnp.int32),
        pltpu.VMEM((_GATHER,), jnp.float32),
        pltpu.VMEM((_GATHER,), jnp.float32),
        pltpu.VMEM((_BPW,), jnp.float32),
        pltpu.SemaphoreType.DMA,
    ],
    compiler_params=pltpu.CompilerParams(needs_layout_passes=False),
)
def _mf_kernel(s_idx_hbm, u_idx_hbm, s_tab_hbm, u_tab_hbm, out_hbm,
               s_idx_v, u_idx_v, s_flat_v, u_flat_v, s_vals_v, u_vals_v,
               out_v, sem):
    wid = lax.axis_index("s") * _NUM_CORES + lax.axis_index("c")
    base = wid * _BPW
    pltpu.sync_copy(s_idx_hbm.at[pl.ds(base, _BPW)], s_idx_v)
    pltpu.sync_copy(u_idx_hbm.at[pl.ds(base, _BPW)], u_idx_v)

    def build(k, carry):
        sl = pl.ds(k * _LANES, _LANES)
        si = s_idx_v[sl]
        ui = u_idx_v[sl]
        for d in range(_DIM):
            dst = pl.ds(d * _BPW + k * _LANES, _LANES)
            s_flat_v[dst] = si + d * _STUDENT_ROWS
            u_flat_v[dst] = ui + d * _SUBJECT_ROWS
        return carry

    lax.fori_loop(0, _BPW // _LANES, build, 0)

    g1 = pltpu.async_copy(s_tab_hbm.at[s_flat_v], s_vals_v, sem)
    g2 = pltpu.async_copy(u_tab_hbm.at[u_flat_v], u_vals_v, sem)
    g1.wait()
    g2.wait()

    def dot(g, carry):
        acc = jnp.zeros((_LANES,), jnp.float32)
        for d in range(_DIM):
            sl = pl.ds(d * _BPW + g * _LANES, _LANES)
            acc = acc + s_vals_v[sl] * u_vals_v[sl]
        out_v[pl.ds(g * _LANES, _LANES)] = acc
        return carry

    lax.fori_loop(0, _BPW // _LANES, dot, 0)
    pltpu.sync_copy(out_v, out_hbm.at[pl.ds(base, _BPW)])


def kernel(student_idx, subject_idx, student_table, subject_table):
    return _mf_kernel(
        student_idx,
        subject_idx,
        student_table.T.reshape(-1),
        subject_table.T.reshape(-1),
    )


# speedup vs baseline: 1.3373x; 1.3373x over previous
import functools

import jax
import jax.numpy as jnp
from jax import lax
from jax.experimental import pallas as pl
from jax.experimental.pallas import tpu as pltpu
from jax.experimental.pallas import tpu_sc as plsc

_BATCH = 16384
_DIM = 16
_LANES = 16
_NUM_CORES = 2
_NUM_SUBCORES = 16
_NW = _NUM_CORES * _NUM_SUBCORES
_BPW = _BATCH // _NW
_GATHER = _BPW * _DIM

_STUDENT_ROWS = 1000000
_SUBJECT_ROWS = 100000

_mesh = plsc.VectorSubcoreMesh(core_axis_name="c", subcore_axis_name="s")


@functools.partial(
    pl.kernel,
    out_type=jax.ShapeDtypeStruct((_BATCH,), jnp.float32),
    mesh=_mesh,
    scratch_types=[
        pltpu.VMEM((_BPW,), jnp.int32),
        pltpu.VMEM((_BPW,), jnp.int32),
        pltpu.VMEM((_GATHER,), jnp.int32),
        pltpu.VMEM((_GATHER,), jnp.int32),
        pltpu.VMEM((_GATHER,), jnp.float32),
        pltpu.VMEM((_GATHER,), jnp.float32),
        pltpu.VMEM((_BPW,), jnp.float32),
        pltpu.SemaphoreType.DMA,
    ],
    compiler_params=pltpu.CompilerParams(needs_layout_passes=False),
)
def _mf_kernel(s_idx_hbm, u_idx_hbm, s_tab_hbm, u_tab_hbm, out_hbm,
               s_idx_v, u_idx_v, s_flat_v, u_flat_v, s_vals_v, u_vals_v,
               out_v, sem):
    wid = lax.axis_index("s") * _NUM_CORES + lax.axis_index("c")
    base = wid * _BPW
    pltpu.sync_copy(s_idx_hbm.at[pl.ds(base, _BPW)], s_idx_v)
    pltpu.sync_copy(u_idx_hbm.at[pl.ds(base, _BPW)], u_idx_v)

    def build(k, carry):
        sl = pl.ds(k * _LANES, _LANES)
        si = s_idx_v[sl]
        ui = u_idx_v[sl]
        for d in range(_DIM):
            dst = pl.ds(d * _BPW + k * _LANES, _LANES)
            s_flat_v[dst] = si + d * _STUDENT_ROWS
            u_flat_v[dst] = ui + d * _SUBJECT_ROWS
        return carry

    lax.fori_loop(0, _BPW // _LANES, build, 0)

    g1 = pltpu.async_copy(s_tab_hbm.at[s_flat_v], s_vals_v, sem)
    g2 = pltpu.async_copy(u_tab_hbm.at[u_flat_v], u_vals_v, sem)
    g1.wait()
    g2.wait()

    def dot(g, carry):
        acc = jnp.zeros((_LANES,), jnp.float32)
        for d in range(_DIM):
            sl = pl.ds(d * _BPW + g * _LANES, _LANES)
            acc = acc + s_vals_v[sl] * u_vals_v[sl]
        out_v[pl.ds(g * _LANES, _LANES)] = acc
        return carry

    lax.fori_loop(0, _BPW // _LANES, dot, 0)
    pltpu.sync_copy(out_v, out_hbm.at[pl.ds(base, _BPW)])


def kernel(student_idx, subject_idx, student_table, subject_table):
    s_flat = jnp.concatenate([student_table[:, d] for d in range(_DIM)])
    u_flat = jnp.concatenate([subject_table[:, d] for d in range(_DIM)])
    return _mf_kernel(student_idx, subject_idx, s_flat, u_flat)
